# baseline (speedup 1.0000x reference)
import functools

import jax
import jax.numpy as jnp
from jax import lax
from jax.experimental import pallas as pl
from jax.experimental.pallas import tpu as pltpu
from jax.experimental.pallas import tpu_sc as plsc

_NC = 2
_NS = 16
_NW = _NC * _NS
_L = 16


def _sc_embed_sum(xi, xo, table_pk, B, HIST, D):
    b_per_w = B // _NW
    CH0 = 104
    CH1 = HIST - CH0
    nd = D // _L
    WP = table_pk.shape[1]
    mesh = plsc.VectorSubcoreMesh(core_axis_name="c", subcore_axis_name="s")

    @functools.partial(
        pl.kernel,
        out_type=jax.ShapeDtypeStruct((B, D), jnp.float32),
        mesh=mesh,
        scratch_types=[
            pltpu.VMEM((b_per_w * HIST,), jnp.int32),
            pltpu.VMEM((b_per_w * HIST,), jnp.int32),
            pltpu.VMEM((HIST, WP), jnp.float32),
            pltpu.VMEM((HIST, WP), jnp.float32),
            pltpu.VMEM((b_per_w, D), jnp.float32),
            pltpu.SemaphoreType.DMA,
            pltpu.SemaphoreType.DMA,
        ],
    )
    def k(x_hbm, xo_hbm, tab_hbm, out_hbm, idx_v, xo_v, rows_a, rows_b,
          sums_v, sem_a, sem_b):
        wid = lax.axis_index("s") * _NC + lax.axis_index("c")
        base = pl.multiple_of(wid * b_per_w, b_per_w)
        pltpu.sync_copy(x_hbm.at[pl.ds(base * HIST, b_per_w * HIST)], idx_v)
        pltpu.sync_copy(xo_hbm.at[pl.ds(base * HIST, b_per_w * HIST)], xo_v)

        def issue(buf, sem, r):
            rh = pl.multiple_of(r * HIST, 8)
            pltpu.async_copy(
                tab_hbm.at[idx_v.at[pl.ds(rh, CH0)]],
                buf.at[pl.ds(0, CH0)], sem)
            pltpu.async_copy(
                tab_hbm.at[idx_v.at[pl.ds(rh + CH0, CH1)]],
                buf.at[pl.ds(CH0, CH1)], sem)

        def wait(buf, sem):
            pltpu.make_async_copy(tab_hbm.at[pl.ds(0, HIST)], buf, sem).wait()

        def acc(buf, r):
            rh = r * HIST
            n16 = HIST // _L

            def step(accs, h, p):
                return tuple(
                    accs[d] + buf[h, pl.ds(p + d * _L, _L)]
                    for d in range(nd))

            def body(g, accs):
                h16 = g * _L
                pv = xo_v[pl.ds(rh + h16, _L)]
                for j in range(_L):
                    accs = step(accs, h16 + j, pv[j])
                return accs

            accs = lax.fori_loop(
                0, n16, body,
                tuple(jnp.zeros((_L,), jnp.float32) for _ in range(nd)))
            pv = xo_v[pl.ds(rh + HIST - _L, _L)]
            for j in range(n16 * _L - (HIST - _L), _L):
                accs = step(accs, HIST - _L + j, pv[j])
            for d in range(nd):
                sums_v[r, pl.ds(d * _L, _L)] = accs[d]

        issue(rows_a, sem_a, 0)

        @pl.loop(0, b_per_w - 2, step=2)
        def _(r):
            issue(rows_b, sem_b, r + 1)
            wait(rows_a, sem_a)
            acc(rows_a, r)
            issue(rows_a, sem_a, r + 2)
            wait(rows_b, sem_b)
            acc(rows_b, r + 1)

        issue(rows_b, sem_b, b_per_w - 1)
        wait(rows_a, sem_a)
        acc(rows_a, b_per_w - 2)
        wait(rows_b, sem_b)
        acc(rows_b, b_per_w - 1)

        pltpu.sync_copy(sums_v, out_hbm.at[pl.ds(base, b_per_w)])

    return k(xi, xo, table_pk)


def _mlp(sums, W1, b1, W2, b2, HIST):
    B, D = sums.shape
    H = W1.shape[1]
    O = W2.shape[1]

    def mlp_body(s_ref, w1_ref, b1_ref, w2_ref, b2_ref, o_ref):
        xm = s_ref[...] * (1.0 / HIST)
        x1 = jnp.dot(xm, w1_ref[...], preferred_element_type=jnp.float32)
        a1 = jnp.maximum(x1 + b1_ref[...], 0.0)
        o_ref[...] = (
            jnp.dot(a1, w2_ref[...], preferred_element_type=jnp.float32)
            + b2_ref[...])

    return pl.pallas_call(
        mlp_body,
        out_shape=jax.ShapeDtypeStruct((B, O), jnp.float32),
    )(sums, W1, b1.reshape(1, H), W2, b2.reshape(1, O))


def kernel(x, table, W1, b1, W2, b2):
    B, HIST = x.shape
    V, D = table.shape
    xi32 = x.astype(jnp.int32)
    xi = (xi32 >> 1).reshape(-1)
    xo = ((xi32 & 1) * D).reshape(-1)
    T3 = table.T.reshape(D, V // 2, 2)
    sel = jnp.zeros((D, 2, 2 * D), jnp.float32)
    sel = sel.at[:, 0, :D].set(jnp.eye(D, dtype=jnp.float32))
    sel = sel.at[:, 1, D:].set(jnp.eye(D, dtype=jnp.float32))
    table_pk = jnp.einsum(
        'ckp,cpj->kj', T3, sel, precision=jax.lax.Precision.DEFAULT)
    sums = _sc_embed_sum(xi, xo, table_pk, B, HIST, D)
    return _mlp(sums, W1, b1, W2, b2, HIST)


# speedup vs baseline: 1.7687x; 1.7687x over previous
import functools

import jax
import jax.numpy as jnp
from jax import lax
from jax.experimental import pallas as pl
from jax.experimental.pallas import tpu as pltpu
from jax.experimental.pallas import tpu_sc as plsc

_NC = 2
_NS = 16
_NW = _NC * _NS
_L = 16


def _sc_embed_sum(xi, table_pad, B, HIST, D):
    b_per_w = B // _NW
    CH0 = 104
    CH1 = HIST - CH0
    nd = D // _L
    WP = table_pad.shape[1]
    mesh = plsc.VectorSubcoreMesh(core_axis_name="c", subcore_axis_name="s")

    @functools.partial(
        pl.kernel,
        out_type=jax.ShapeDtypeStruct((B, D), jnp.float32),
        mesh=mesh,
        scratch_types=[
            pltpu.VMEM((b_per_w * HIST,), jnp.int32),
            pltpu.VMEM((HIST, WP), jnp.float32),
            pltpu.VMEM((HIST, WP), jnp.float32),
            pltpu.VMEM((HIST, WP), jnp.float32),
            pltpu.VMEM((b_per_w, D), jnp.float32),
            pltpu.SemaphoreType.DMA,
            pltpu.SemaphoreType.DMA,
            pltpu.SemaphoreType.DMA,
        ],
        compiler_params=pltpu.CompilerParams(use_tc_tiling_on_sc=False),
    )
    def k(x_hbm, tab_hbm, out_hbm, idx_v, rows_a, rows_b, rows_c, sums_v,
          sem_a, sem_b, sem_c):
        wid = lax.axis_index("s") * _NC + lax.axis_index("c")
        base = pl.multiple_of(wid * b_per_w, b_per_w)
        pltpu.sync_copy(x_hbm.at[pl.ds(base * HIST, b_per_w * HIST)], idx_v)

        def issue(buf, sem, r):
            rh = pl.multiple_of(r * HIST, 8)
            pltpu.async_copy(
                tab_hbm.at[idx_v.at[pl.ds(rh, CH0)]],
                buf.at[pl.ds(0, CH0)], sem)
            pltpu.async_copy(
                tab_hbm.at[idx_v.at[pl.ds(rh + CH0, CH1)]],
                buf.at[pl.ds(CH0, CH1)], sem)

        def wait(buf, sem):
            pltpu.make_async_copy(tab_hbm.at[pl.ds(0, HIST)], buf, sem).wait()

        def acc(buf, r):
            def body(h, accs):
                return tuple(
                    accs[d] + buf[h, pl.ds(d * _L, _L)] for d in range(nd))

            accs = lax.fori_loop(
                0, HIST, body,
                tuple(jnp.zeros((_L,), jnp.float32) for _ in range(nd)),
                unroll=4)
            for d in range(nd):
                sums_v[r, pl.ds(d * _L, _L)] = accs[d]

        issue(rows_a, sem_a, 0)
        issue(rows_b, sem_b, 1)

        @pl.loop(0, b_per_w - 2, step=3)
        def _(r):
            issue(rows_c, sem_c, r + 2)
            wait(rows_a, sem_a)
            acc(rows_a, r)
            issue(rows_a, sem_a, r + 3)
            wait(rows_b, sem_b)
            acc(rows_b, r + 1)
            issue(rows_b, sem_b, r + 4)
            wait(rows_c, sem_c)
            acc(rows_c, r + 2)

        wait(rows_a, sem_a)
        acc(rows_a, b_per_w - 2)
        wait(rows_b, sem_b)
        acc(rows_b, b_per_w - 1)

        pltpu.sync_copy(sums_v, out_hbm.at[pl.ds(base, b_per_w)])

    return k(xi, table_pad)


def _mlp(sums, W1, b1, W2, b2, HIST):
    B, D = sums.shape
    H = W1.shape[1]
    O = W2.shape[1]

    def mlp_body(s_ref, w1_ref, b1_ref, w2_ref, b2_ref, o_ref):
        xm = s_ref[...] * (1.0 / HIST)
        x1 = jnp.dot(xm, w1_ref[...], preferred_element_type=jnp.float32)
        a1 = jnp.maximum(x1 + b1_ref[...], 0.0)
        o_ref[...] = (
            jnp.dot(a1, w2_ref[...], preferred_element_type=jnp.float32)
            + b2_ref[...])

    return pl.pallas_call(
        mlp_body,
        out_shape=jax.ShapeDtypeStruct((B, O), jnp.float32),
    )(sums, W1, b1.reshape(1, H), W2, b2.reshape(1, O))


def kernel(x, table, W1, b1, W2, b2):
    B, HIST = x.shape
    _, D = table.shape
    V = table.shape[0]
    xi = x.astype(jnp.int32).reshape(-1)
    T3 = table.T.reshape(D, V // 2, 2)
    sel = jnp.zeros((D, 2, 2 * D), jnp.float32)
    sel = sel.at[:, 0, :D].set(jnp.eye(D, dtype=jnp.float32))
    sel = sel.at[:, 1, D:].set(jnp.eye(D, dtype=jnp.float32))
    table_pk = jnp.einsum(
        'ckp,cpj->kj', T3, sel, precision=jax.lax.Precision.DEFAULT)
    table_lin = table_pk.reshape(V, D)
    sums = _sc_embed_sum(xi, table_lin, B, HIST, D)
    return _mlp(sums, W1, b1, W2, b2, HIST)


# speedup vs baseline: 2.7211x; 1.5384x over previous
import functools

import jax
import jax.numpy as jnp
from jax import lax
from jax.experimental import pallas as pl
from jax.experimental.pallas import tpu as pltpu
from jax.experimental.pallas import tpu_sc as plsc

_NC = 2
_NS = 16
_NW = _NC * _NS
_L = 16


def _sc_embed_sum(xi, table_pad, B, HIST, D):
    b_per_w = B // _NW
    CH0 = 104
    CH1 = HIST - CH0
    nd = D // _L
    WP = table_pad.shape[1]
    mesh = plsc.VectorSubcoreMesh(core_axis_name="c", subcore_axis_name="s")

    @functools.partial(
        pl.kernel,
        out_type=jax.ShapeDtypeStruct((B, D), jnp.float32),
        mesh=mesh,
        scratch_types=[
            pltpu.VMEM((b_per_w * HIST,), jnp.int32),
            pltpu.VMEM((HIST, WP), jnp.float32),
            pltpu.VMEM((HIST, WP), jnp.float32),
            pltpu.VMEM((HIST, WP), jnp.float32),
            pltpu.VMEM((b_per_w, D), jnp.float32),
            pltpu.SemaphoreType.DMA,
            pltpu.SemaphoreType.DMA,
            pltpu.SemaphoreType.DMA,
        ],
        compiler_params=pltpu.CompilerParams(use_tc_tiling_on_sc=False),
    )
    def k(x_hbm, tab_hbm, out_hbm, idx_v, rows_a, rows_b, rows_c, sums_v,
          sem_a, sem_b, sem_c):
        wid = lax.axis_index("s") * _NC + lax.axis_index("c")
        base = pl.multiple_of(wid * b_per_w, b_per_w)
        pltpu.sync_copy(x_hbm.at[pl.ds(base * HIST, b_per_w * HIST)], idx_v)

        def issue(buf, sem, r):
            rh = pl.multiple_of(r * HIST, 8)
            pltpu.async_copy(
                tab_hbm.at[idx_v.at[pl.ds(rh, CH0)]],
                buf.at[pl.ds(0, CH0)], sem)
            pltpu.async_copy(
                tab_hbm.at[idx_v.at[pl.ds(rh + CH0, CH1)]],
                buf.at[pl.ds(CH0, CH1)], sem)

        def wait(buf, sem):
            pltpu.make_async_copy(tab_hbm.at[pl.ds(0, HIST)], buf, sem).wait()

        def acc(buf, r):
            def body(h, accs):
                return tuple(
                    accs[d] + buf[h, pl.ds(d * _L, _L)] for d in range(nd))

            accs = lax.fori_loop(
                0, HIST, body,
                tuple(jnp.zeros((_L,), jnp.float32) for _ in range(nd)),
                unroll=4)
            for d in range(nd):
                sums_v[r, pl.ds(d * _L, _L)] = accs[d]

        issue(rows_a, sem_a, 0)
        issue(rows_b, sem_b, 1)

        @pl.loop(0, b_per_w - 2, step=3)
        def _(r):
            issue(rows_c, sem_c, r + 2)
            wait(rows_a, sem_a)
            acc(rows_a, r)
            issue(rows_a, sem_a, r + 3)
            wait(rows_b, sem_b)
            acc(rows_b, r + 1)
            issue(rows_b, sem_b, r + 4)
            wait(rows_c, sem_c)
            acc(rows_c, r + 2)

        wait(rows_a, sem_a)
        acc(rows_a, b_per_w - 2)
        wait(rows_b, sem_b)
        acc(rows_b, b_per_w - 1)

        pltpu.sync_copy(sums_v, out_hbm.at[pl.ds(base, b_per_w)])

    return k(xi, table_pad)


def _mlp(sums, W1, b1, W2, b2, HIST):
    B, D = sums.shape
    H = W1.shape[1]
    O = W2.shape[1]

    def mlp_body(s_ref, w1_ref, b1_ref, w2_ref, b2_ref, o_ref):
        xm = s_ref[...] * (1.0 / HIST)
        x1 = jnp.dot(xm, w1_ref[...], preferred_element_type=jnp.float32)
        a1 = jnp.maximum(x1 + b1_ref[...], 0.0)
        o_ref[...] = (
            jnp.dot(a1, w2_ref[...], preferred_element_type=jnp.float32)
            + b2_ref[...])

    return pl.pallas_call(
        mlp_body,
        out_shape=jax.ShapeDtypeStruct((B, O), jnp.float32),
    )(sums, W1, b1.reshape(1, H), W2, b2.reshape(1, O))


def kernel(x, table, W1, b1, W2, b2):
    B, HIST = x.shape
    _, D = table.shape
    V = table.shape[0]
    xi = (x.astype(jnp.int32) << 1).reshape(-1)
    sel = jnp.concatenate(
        [jnp.eye(D, dtype=jnp.float32),
         jnp.zeros((D, 128 - D), jnp.float32)], axis=1)
    table_pad = jax.lax.dot_general(
        table.T, sel, dimension_numbers=(((0,), (0,)), ((), ())),
        precision=jax.lax.Precision.DEFAULT)
    table_lin = table_pad.reshape(2 * V, D)
    sums = _sc_embed_sum(xi, table_lin, B, HIST, D)
    return _mlp(sums, W1, b1, W2, b2, HIST)


# speedup vs baseline: 2.7870x; 1.0242x over previous
import functools

import jax
import jax.numpy as jnp
from jax import lax
from jax.experimental import pallas as pl
from jax.experimental.pallas import tpu as pltpu
from jax.experimental.pallas import tpu_sc as plsc

_NC = 2
_NS = 16
_NW = _NC * _NS
_L = 16


def _sc_embed_sum(xi, table_pad, B, HIST, D):
    b_per_w = B // _NW
    CH0 = 104
    CH1 = HIST - CH0
    nd = D // _L
    WP = table_pad.shape[1]
    mesh = plsc.VectorSubcoreMesh(core_axis_name="c", subcore_axis_name="s")

    @functools.partial(
        pl.kernel,
        out_type=jax.ShapeDtypeStruct((B, D), jnp.float32),
        mesh=mesh,
        scratch_types=[
            pltpu.VMEM((b_per_w * HIST,), jnp.int32),
            pltpu.VMEM((HIST, WP), jnp.float32),
            pltpu.VMEM((HIST, WP), jnp.float32),
            pltpu.VMEM((HIST, WP), jnp.float32),
            pltpu.VMEM((HIST, WP), jnp.float32),
            pltpu.VMEM((b_per_w, D), jnp.float32),
            pltpu.SemaphoreType.DMA,
            pltpu.SemaphoreType.DMA,
            pltpu.SemaphoreType.DMA,
            pltpu.SemaphoreType.DMA,
        ],
        compiler_params=pltpu.CompilerParams(use_tc_tiling_on_sc=False),
    )
    def k(x_hbm, tab_hbm, out_hbm, idx_v, rows_a, rows_b, rows_c, rows_d,
          sums_v, sem_a, sem_b, sem_c, sem_d):
        wid = lax.axis_index("s") * _NC + lax.axis_index("c")
        base = pl.multiple_of(wid * b_per_w, b_per_w)
        pltpu.sync_copy(x_hbm.at[pl.ds(base * HIST, b_per_w * HIST)], idx_v)

        def issue(buf, sem, r):
            rh = pl.multiple_of(r * HIST, 8)
            pltpu.async_copy(
                tab_hbm.at[idx_v.at[pl.ds(rh, CH0)]],
                buf.at[pl.ds(0, CH0)], sem)
            pltpu.async_copy(
                tab_hbm.at[idx_v.at[pl.ds(rh + CH0, CH1)]],
                buf.at[pl.ds(CH0, CH1)], sem)

        def wait(buf, sem):
            pltpu.make_async_copy(tab_hbm.at[pl.ds(0, HIST)], buf, sem).wait()

        def acc(buf, r):
            def body(h, accs):
                return tuple(
                    accs[d] + buf[h, pl.ds(d * _L, _L)] for d in range(nd))

            accs = lax.fori_loop(
                0, HIST, body,
                tuple(jnp.zeros((_L,), jnp.float32) for _ in range(nd)),
                unroll=4)
            for d in range(nd):
                sums_v[r, pl.ds(d * _L, _L)] = accs[d]

        issue(rows_a, sem_a, 0)
        issue(rows_b, sem_b, 1)
        issue(rows_c, sem_c, 2)

        @pl.loop(0, b_per_w - 4, step=4)
        def _(r):
            issue(rows_d, sem_d, r + 3)
            wait(rows_a, sem_a)
            acc(rows_a, r)
            issue(rows_a, sem_a, r + 4)
            wait(rows_b, sem_b)
            acc(rows_b, r + 1)
            issue(rows_b, sem_b, r + 5)
            wait(rows_c, sem_c)
            acc(rows_c, r + 2)
            issue(rows_c, sem_c, r + 6)
            wait(rows_d, sem_d)
            acc(rows_d, r + 3)

        issue(rows_d, sem_d, b_per_w - 1)
        wait(rows_a, sem_a)
        acc(rows_a, b_per_w - 4)
        wait(rows_b, sem_b)
        acc(rows_b, b_per_w - 3)
        wait(rows_c, sem_c)
        acc(rows_c, b_per_w - 2)
        wait(rows_d, sem_d)
        acc(rows_d, b_per_w - 1)

        pltpu.sync_copy(sums_v, out_hbm.at[pl.ds(base, b_per_w)])

    return k(xi, table_pad)


def _mlp(sums, W1, b1, W2, b2, HIST):
    B, D = sums.shape
    H = W1.shape[1]
    O = W2.shape[1]

    def mlp_body(s_ref, w1_ref, b1_ref, w2_ref, b2_ref, o_ref):
        xm = s_ref[...] * (1.0 / HIST)
        x1 = jnp.dot(xm, w1_ref[...], preferred_element_type=jnp.float32)
        a1 = jnp.maximum(x1 + b1_ref[...], 0.0)
        o_ref[...] = (
            jnp.dot(a1, w2_ref[...], preferred_element_type=jnp.float32)
            + b2_ref[...])

    return pl.pallas_call(
        mlp_body,
        out_shape=jax.ShapeDtypeStruct((B, O), jnp.float32),
    )(sums, W1, b1.reshape(1, H), W2, b2.reshape(1, O))


def kernel(x, table, W1, b1, W2, b2):
    B, HIST = x.shape
    _, D = table.shape
    V = table.shape[0]
    xi = (x.astype(jnp.int32) << 1).reshape(-1)
    sel = jnp.concatenate(
        [jnp.eye(D, dtype=jnp.float32),
         jnp.zeros((D, 128 - D), jnp.float32)], axis=1)
    table_pad = jax.lax.dot_general(
        table.T, sel, dimension_numbers=(((0,), (0,)), ((), ())),
        precision=jax.lax.Precision.DEFAULT)
    table_lin = table_pad.reshape(2 * V, D)
    sums = _sc_embed_sum(xi, table_lin, B, HIST, D)
    return _mlp(sums, W1, b1, W2, b2, HIST)


# speedup vs baseline: 2.7921x; 1.0018x over previous
import functools

import jax
import jax.numpy as jnp
from jax import lax
from jax.experimental import pallas as pl
from jax.experimental.pallas import tpu as pltpu
from jax.experimental.pallas import tpu_sc as plsc

_NC = 2
_NS = 16
_NW = _NC * _NS
_L = 16


def _sc_embed_sum(xi, table_lin, B, HIST, D):
    b_per_w = B // _NW
    CH0 = 104
    CH1 = HIST - CH0
    nd = D // _L
    WP = table_lin.shape[1]
    mesh = plsc.VectorSubcoreMesh(core_axis_name="c", subcore_axis_name="s")

    @functools.partial(
        pl.kernel,
        out_type=jax.ShapeDtypeStruct((B, D), jnp.float32),
        mesh=mesh,
        scratch_types=[
            pltpu.VMEM((b_per_w * HIST,), jnp.int32),
            pltpu.VMEM((HIST, WP), jnp.float32),
            pltpu.VMEM((HIST, WP), jnp.float32),
            pltpu.VMEM((HIST, WP), jnp.float32),
            pltpu.VMEM((HIST, WP), jnp.float32),
            pltpu.VMEM((b_per_w, D), jnp.float32),
            pltpu.SemaphoreType.DMA,
            pltpu.SemaphoreType.DMA,
            pltpu.SemaphoreType.DMA,
            pltpu.SemaphoreType.DMA,
        ],
        compiler_params=pltpu.CompilerParams(use_tc_tiling_on_sc=False),
    )
    def k(x_hbm, tab_hbm, out_hbm, idx_v, rows_a, rows_b, rows_c, rows_d,
          sums_v, sem_a, sem_b, sem_c, sem_d):
        wid = lax.axis_index("s") * _NC + lax.axis_index("c")
        base = pl.multiple_of(wid * b_per_w, b_per_w)
        pltpu.sync_copy(x_hbm.at[pl.ds(base * HIST, b_per_w * HIST)], idx_v)

        def issue(buf, sem, r):
            rh = pl.multiple_of(r * HIST, 8)
            pltpu.async_copy(
                tab_hbm.at[idx_v.at[pl.ds(rh, CH0)]],
                buf.at[pl.ds(0, CH0)], sem)
            pltpu.async_copy(
                tab_hbm.at[idx_v.at[pl.ds(rh + CH0, CH1)]],
                buf.at[pl.ds(CH0, CH1)], sem)

        def wait(buf, sem):
            pltpu.make_async_copy(tab_hbm.at[pl.ds(0, HIST)], buf, sem).wait()

        def acc(buf, r):
            def body(h, accs):
                return tuple(
                    accs[d] + buf[h, pl.ds(d * _L, _L)] for d in range(nd))

            accs = lax.fori_loop(
                0, HIST, body,
                tuple(jnp.zeros((_L,), jnp.float32) for _ in range(nd)),
                unroll=4)
            for d in range(nd):
                sums_v[r, pl.ds(d * _L, _L)] = accs[d]

        issue(rows_a, sem_a, 0)
        issue(rows_b, sem_b, 1)
        issue(rows_c, sem_c, 2)

        @pl.loop(0, b_per_w - 4, step=4)
        def _(r):
            issue(rows_d, sem_d, r + 3)
            wait(rows_a, sem_a)
            acc(rows_a, r)
            issue(rows_a, sem_a, r + 4)
            wait(rows_b, sem_b)
            acc(rows_b, r + 1)
            issue(rows_b, sem_b, r + 5)
            wait(rows_c, sem_c)
            acc(rows_c, r + 2)
            issue(rows_c, sem_c, r + 6)
            wait(rows_d, sem_d)
            acc(rows_d, r + 3)

        issue(rows_d, sem_d, b_per_w - 1)
        wait(rows_a, sem_a)
        acc(rows_a, b_per_w - 4)
        wait(rows_b, sem_b)
        acc(rows_b, b_per_w - 3)
        wait(rows_c, sem_c)
        acc(rows_c, b_per_w - 2)
        wait(rows_d, sem_d)
        acc(rows_d, b_per_w - 1)

        pltpu.sync_copy(sums_v, out_hbm.at[pl.ds(base, b_per_w)])

    return k(xi, table_lin)


def _mlp(sums, W1, b1, W2, b2, HIST):
    B, D = sums.shape
    H = W1.shape[1]
    O = W2.shape[1]

    def mlp_body(s_ref, w1_ref, b1_ref, w2_ref, b2_ref, o_ref):
        xm = s_ref[...] * (1.0 / HIST)
        x1 = jnp.dot(xm, w1_ref[...], preferred_element_type=jnp.float32)
        a1 = jnp.maximum(x1 + b1_ref[...], 0.0)
        o_ref[...] = (
            jnp.dot(a1, w2_ref[...], preferred_element_type=jnp.float32)
            + b2_ref[...])

    return pl.pallas_call(
        mlp_body,
        out_shape=jax.ShapeDtypeStruct((B, O), jnp.float32),
    )(sums, W1, b1.reshape(1, H), W2, b2.reshape(1, O))


def kernel(x, table, W1, b1, W2, b2):
    B, HIST = x.shape
    _, D = table.shape
    V = table.shape[0]
    xi = (x.astype(jnp.int32) << 1).reshape(-1)
    sel = jnp.concatenate(
        [jnp.eye(D, dtype=jnp.float32),
         jnp.zeros((D, 128 - D), jnp.float32)], axis=1)
    table_pad = jax.lax.dot_general(
        table.T, sel, dimension_numbers=(((0,), (0,)), ((), ())),
        precision=jax.lax.Precision.DEFAULT)
    table_lin = table_pad.reshape(2 * V, D)
    sums = _sc_embed_sum(xi, table_lin, B, HIST, D)
    return _mlp(sums, W1, b1, W2, b2, HIST)
